# baseline (speedup 1.0000x reference)
import functools

import jax
import jax.numpy as jnp
from jax import lax
from jax.experimental import pallas as pl
from jax.experimental.pallas import tpu as pltpu
from jax.experimental.pallas import tpu_sc as plsc


def _make_node_gather(B, N, F):
    info = plsc.get_sparse_core_info()
    nc, ns, L = info.num_cores, info.num_subcores, info.num_lanes
    nw = nc * ns
    assert B % (8 * nw) == 0
    bpw = B // nw
    mesh = plsc.VectorSubcoreMesh(core_axis_name="c", subcore_axis_name="s")

    def body(no2, aid, no_row, aid_v, idx_v, nrows_v, sem):
        wid = lax.axis_index("s") * nc + lax.axis_index("c")
        base = wid * bpw
        pltpu.sync_copy(aid.at[pl.ds(base, bpw)], aid_v)
        for j in range(bpw // L):
            ids = aid_v[pl.ds(j * L, L)]
            r_star = ((base + j * L) + lax.iota(jnp.int32, L)) * N + ids
            idx_v[pl.ds(j * L, L)] = r_star
        pltpu.async_copy(no2.at[idx_v], nrows_v, sem).wait()
        pltpu.sync_copy(nrows_v, no_row.at[pl.ds(base, bpw)])

    return pl.kernel(
        body,
        out_type=jax.ShapeDtypeStruct((B, F), jnp.float32),
        mesh=mesh,
        scratch_types=[
            pltpu.VMEM((bpw,), jnp.int32),
            pltpu.VMEM((bpw,), jnp.int32),
            pltpu.VMEM((bpw, F), jnp.float32),
            pltpu.SemaphoreType.DMA,
        ],
        compiler_params=pltpu.CompilerParams(
            use_tc_tiling_on_sc=False, needs_layout_passes=False),
        name="sc_node_row_gather",
    )


def _dense_body(N, adj_ref, aid_ref, nrow_ref, obs_ref, nodes_ref, Wi_ref,
                bi_ref, Wm_ref, Wu_ref, W1_ref, b1_ref, W2_ref, b2_ref,
                Wq_ref, bq_ref, q_ref, xbuf, nsem):
    Bb = aid_ref.shape[0]
    HG = Wi_ref.shape[1]
    i = pl.program_id(0)
    rows = xbuf.shape[0]
    pltpu.make_async_copy(
        nodes_ref.at[pl.ds(i * rows, rows)], xbuf, nsem).start()
    pltpu.make_async_copy(
        nodes_ref.at[pl.ds(i * rows, rows)], xbuf, nsem).wait()
    x = xbuf[...]
    Wi = Wi_ref[...]
    bi = bi_ref[...]
    h = jnp.maximum(jnp.dot(x, Wi, preferred_element_type=jnp.float32) + bi, 0.0)
    msg = jnp.maximum(jnp.dot(h, Wm_ref[...], preferred_element_type=jnp.float32), 0.0)
    msg3 = msg.reshape(Bb, N, HG)
    aid_i = aid_ref[...]
    node_iota = lax.broadcasted_iota(jnp.int32, (Bb, N), 1)
    onehot = jnp.where(node_iota == aid_i, 1.0, 0.0)
    w = jnp.sum(adj_ref[...] * onehot[:, :, None], axis=1)
    w = jnp.where(w > 0.0, w, 0.0)
    agg = jnp.sum(msg3 * w[:, :, None], axis=1)
    hrow = jnp.maximum(
        jnp.dot(nrow_ref[...], Wi, preferred_element_type=jnp.float32) + bi, 0.0)
    h2 = jnp.maximum(
        hrow + jnp.dot(agg, Wu_ref[...], preferred_element_type=jnp.float32), 0.0)
    inp = jnp.concatenate([obs_ref[...], h2], axis=1)
    z = jnp.maximum(
        jnp.dot(inp, W1_ref[...], preferred_element_type=jnp.float32) + b1_ref[...], 0.0)
    z = jnp.maximum(
        jnp.dot(z, W2_ref[...], preferred_element_type=jnp.float32) + b2_ref[...], 0.0)
    q_ref[...] = jnp.dot(z, Wq_ref[...], preferred_element_type=jnp.float32) + bq_ref[...]


def kernel(obs, rnn_states, node_obs, adj, agent_id, W_in, b_in, W_msg,
           W_upd, W1, b1, W2, b2, Wq, bq):
    B, N, F = node_obs.shape
    OBS = obs.shape[1]
    HG = W_in.shape[1]
    HID = W1.shape[1]
    ACT = Wq.shape[1]

    aid = agent_id.reshape(B).astype(jnp.int32)
    no2 = node_obs.reshape(B * N, F)
    no_row = _make_node_gather(B, N, F)(no2, aid)
    aid_f = aid.reshape(B, 1)

    Bb = 128
    grid = (B // Bb,)
    full = lambda shape: pl.BlockSpec(shape, lambda i: (0,) * len(shape))
    q = pl.pallas_call(
        functools.partial(_dense_body, N),
        grid=grid,
        in_specs=[
            pl.BlockSpec((Bb, N, N), lambda i: (i, 0, 0)),
            pl.BlockSpec((Bb, 1), lambda i: (i, 0)),
            pl.BlockSpec((Bb, F), lambda i: (i, 0)),
            pl.BlockSpec((Bb, OBS), lambda i: (i, 0)),
            pl.BlockSpec(memory_space=pltpu.MemorySpace.HBM),
            full((F, HG)),
            full((1, HG)),
            full((HG, HG)),
            full((HG, HG)),
            full((OBS + HG, HID)),
            full((1, HID)),
            full((HID, HID)),
            full((1, HID)),
            full((HID, ACT)),
            full((1, ACT)),
        ],
        out_specs=pl.BlockSpec((Bb, ACT), lambda i: (i, 0)),
        out_shape=jax.ShapeDtypeStruct((B, ACT), jnp.float32),
        scratch_shapes=[
            pltpu.VMEM((Bb * N, F), jnp.float32),
            pltpu.SemaphoreType.DMA,
        ],
    )(adj, aid_f, no_row, obs, no2, W_in, b_in.reshape(1, HG), W_msg, W_upd,
      W1, b1.reshape(1, HID), W2, b2.reshape(1, HID), Wq, bq.reshape(1, ACT))

    return (q, rnn_states)
